# baseline (speedup 1.0000x reference)
import functools

import jax
import jax.numpy as jnp
from jax import lax
from jax.experimental import pallas as pl
from jax.experimental.pallas import tpu as pltpu
from jax.experimental.pallas import tpu_sc as plsc

N_CLASSES = 5
K = 32 * N_CLASSES
D = 192
HW = 384 * 384
B = 2
PB = 3072
NB = HW // PB
IGNORE_LB = 255


def _i0():
    return jnp.int32(0)


NW = 32
CH = B * HW // NW


def _sc_hist_body(lab_hbm, idx_hbm, out_hbm, lab_v, idx_v, acc_v):
    wid = lax.axis_index("s") * jnp.int32(2) + lax.axis_index("c")
    base = wid * jnp.int32(CH)
    pltpu.sync_copy(lab_hbm.at[pl.ds(base, CH)], lab_v)
    pltpu.sync_copy(idx_hbm.at[pl.ds(base, CH)], idx_v)
    zeros16 = jnp.zeros((16,), jnp.float32)
    for t in range(K // 16):
        acc_v[pl.ds(t * 16, 16)] = zeros16
    ones16 = jnp.ones((16,), jnp.float32)

    def body(i, carry):
        off = i * jnp.int32(16)
        l16 = lab_v[pl.ds(off, 16)]
        i16 = idx_v[pl.ds(off, 16)]
        k2 = jnp.where(l16 == jnp.int32(IGNORE_LB),
                       jnp.int32(0), l16 * jnp.int32(32) + i16)
        plsc.addupdate_scatter(acc_v, [k2], ones16)
        return carry

    lax.fori_loop(jnp.int32(0), jnp.int32(CH // 16), body, jnp.int32(0))
    pltpu.sync_copy(acc_v, out_hbm.at[wid])


@functools.partial(
    pl.kernel,
    mesh=plsc.VectorSubcoreMesh(core_axis_name="c", subcore_axis_name="s"),
    out_type=jax.ShapeDtypeStruct((NW, K), jnp.float32),
    scratch_types=[
        pltpu.VMEM((CH,), jnp.int32),
        pltpu.VMEM((CH,), jnp.int32),
        pltpu.VMEM((K,), jnp.float32),
    ],
    compiler_params=pltpu.CompilerParams(needs_layout_passes=False),
)
def _sc_hist(lab_hbm, idx_hbm, out_hbm, lab_v, idx_v, acc_v):
    _sc_hist_body(lab_hbm, idx_hbm, out_hbm, lab_v, idx_v, acc_v)


def _segsum_body(lab_ref, idx_ref, feat_ref, sums_ref):
    j = pl.program_id(1)

    @pl.when(j == 0)
    def _init():
        sums_ref[...] = jnp.zeros_like(sums_ref)

    lab = lab_ref[0].reshape(1, PB)
    idx = idx_ref[0].reshape(1, PB)
    k2 = jnp.where(lab == IGNORE_LB, 0, lab * 32 + idx)
    rows = lax.broadcasted_iota(jnp.int32, (K, PB), 0)
    oh_t = (rows == jnp.broadcast_to(k2, (K, PB))).astype(jnp.float32)
    feat = feat_ref[0].reshape(D, PB)
    dn = (((1,), (1,)), ((), ()))
    acc = lax.dot_general(feat, oh_t, dn, preferred_element_type=jnp.float32)
    sums_ref[...] += acc[None]


def _finalize_body(sums_ref, counts_ref, out_ref):
    f32 = jnp.float32
    KK = (K, K)
    row = lax.broadcasted_iota(jnp.int32, KK, 0)
    col = lax.broadcasted_iota(jnp.int32, KK, 1)
    eye = row == col
    ks = lax.broadcasted_iota(jnp.int32, (1, K), 1)

    def _col(x, zero):
        return jnp.sum(jnp.where(eye, jnp.broadcast_to(x, KK), zero),
                       axis=1, keepdims=True, dtype=x.dtype)

    total = f32(0.0)
    n_valid = f32(0.0)

    for b in range(B):
        c2 = jnp.sum(counts_ref[b], axis=0, keepdims=True)
        s2 = sums_ref[b]
        mx = jnp.max(jnp.where(c2 > 0, jnp.bitwise_and(ks, 31), 0))
        mx_safe = jnp.maximum(mx, 1)
        keyed = mx * jnp.right_shift(row, 5) + jnp.bitwise_and(row, 31)
        remap = (keyed == col).astype(f32)
        counts = jnp.dot(c2, remap, preferred_element_type=f32)
        sums = jnp.dot(s2, remap, preferred_element_type=f32)
        means = sums / jnp.maximum(counts, 1.0)

        present = counts > 0
        n_present = jnp.sum(present.astype(f32))

        P = jnp.zeros(KK, f32)
        for db in range(D // 8):
            md = means[db * 8:(db + 1) * 8, :]
            diff = jnp.abs(md[:, :, None] - md[:, None, :])
            P = P + jnp.sum(diff, axis=0)
        P = P * f32(1.0 / D)

        pk = jnp.where(present, ks, -1)
        k_last = jnp.max(pk)
        k_prev = jnp.max(jnp.where(ks == k_last, -1, pk))
        mxf = mx_safe.astype(f32)
        cls_f = jnp.floor((ks.astype(f32) - 1.0) / mxf)
        eff = jnp.where(ks == k_last,
                        jnp.floor((k_prev.astype(f32) - 1.0) / mxf),
                        cls_f)
        included = present & (counts >= 2.0) & (ks >= 1)
        incval = jnp.where(included, ks, -1)
        inc_col = _col(incval, 0)
        strict = row < col
        prev_idx = jnp.max(jnp.where(strict, jnp.broadcast_to(inc_col, KK), -1),
                           axis=0, keepdims=True)
        pidx = jnp.maximum(prev_idx, 0)
        eff_col = _col(eff, f32(0.0))
        gmat = row == jnp.broadcast_to(pidx, KK)
        prev_cls = jnp.sum(jnp.where(gmat, jnp.broadcast_to(eff_col, KK), 0.0),
                           axis=0, keepdims=True)
        prev_cls = jnp.where(prev_idx < 0, f32(-1e9), prev_cls)
        new_group = included & (eff != prev_cls)
        ng_col = _col(new_group.astype(f32), f32(0.0))
        lower = row <= col
        cums = jnp.sum(jnp.where(lower, jnp.broadcast_to(ng_col, KK), 0.0),
                       axis=0, keepdims=True)
        gid = jnp.where(included, cums - 1.0, f32(-1.0))
        memb = (row.astype(f32) == jnp.broadcast_to(gid, KK)).astype(f32)
        sizes = jnp.sum(memb, axis=1, keepdims=True)
        mp = jnp.dot(memb, P, preferred_element_type=f32)
        num = jnp.sum(mp * memb, axis=1, keepdims=True)
        ret = num / (jnp.maximum(sizes, 1.0) ** 2)
        ret = jnp.where(ret < 1.0, 0.5 * ret * ret, ret - 0.5)
        validg = (sizes > 0.0) & (n_present > 1.0)
        total = total + jnp.sum(jnp.where(validg, ret, 0.0))
        n_valid = n_valid + jnp.sum(validg.astype(f32))

    loss = jnp.where(n_valid > 0.0, total / jnp.maximum(n_valid, 1.0), 0.0)
    out_ref[...] = jnp.broadcast_to(loss * f32(1.0 / B), (1, 1))


def kernel(feature_out, labels, indexes):
    lab = labels.astype(jnp.int32)
    idx = indexes.astype(jnp.int32)
    counts = _sc_hist(lab.reshape(B * HW), idx.reshape(B * HW))
    counts = counts.reshape(B, NW // B, K)
    sums = pl.pallas_call(
        _segsum_body,
        grid=(B, NB),
        in_specs=[
            pl.BlockSpec((1, 8, 384), lambda b, j: (b, j, _i0())),
            pl.BlockSpec((1, 8, 384), lambda b, j: (b, j, _i0())),
            pl.BlockSpec((1, D, 8, 384), lambda b, j: (b, _i0(), j, _i0())),
        ],
        out_specs=pl.BlockSpec((1, D, K), lambda b, j: (b, _i0(), _i0())),
        out_shape=jax.ShapeDtypeStruct((B, D, K), jnp.float32),
        compiler_params=pltpu.CompilerParams(
            dimension_semantics=("arbitrary", "arbitrary")),
    )(lab, idx, feature_out)
    loss = pl.pallas_call(
        _finalize_body,
        out_shape=jax.ShapeDtypeStruct((1, 1), jnp.float32),
    )(sums, counts)
    return loss.reshape(1)


# speedup vs baseline: 1.0766x; 1.0766x over previous
import jax
import jax.numpy as jnp
from jax import lax
from jax.experimental import pallas as pl
from jax.experimental.pallas import tpu as pltpu

N_CLASSES = 5
K = 32 * N_CLASSES
D = 192
HW = 384 * 384
B = 2
PB = 3072
NB = HW // PB
IGNORE_LB = 255


def _i0():
    return jnp.int32(0)


def _segsum_body(lab_ref, idx_ref, feat_ref, sums_ref, counts_ref):
    j = pl.program_id(1)

    @pl.when(j == 0)
    def _init():
        sums_ref[...] = jnp.zeros_like(sums_ref)
        counts_ref[...] = jnp.zeros_like(counts_ref)

    lab = lab_ref[0].reshape(1, PB)
    idx = idx_ref[0].reshape(1, PB)
    k2 = jnp.where(lab == IGNORE_LB, 0, lab * 32 + idx)
    rows = lax.broadcasted_iota(jnp.int32, (K, PB), 0)
    oh_t = (rows == jnp.broadcast_to(k2, (K, PB))).astype(jnp.float32)
    feat = feat_ref[0].reshape(D, PB)
    dn = (((1,), (1,)), ((), ()))
    acc = lax.dot_general(feat, oh_t, dn, preferred_element_type=jnp.float32)
    sums_ref[...] += acc[None]
    cnt = oh_t[:, 0:128]
    for c in range(1, PB // 128):
        cnt = cnt + oh_t[:, c * 128:(c + 1) * 128]
    counts_ref[...] += cnt[None]


def _finalize_body(sums_ref, counts_ref, out_ref):
    f32 = jnp.float32
    KK = (K, K)
    row = lax.broadcasted_iota(jnp.int32, KK, 0)
    col = lax.broadcasted_iota(jnp.int32, KK, 1)
    eye = row == col
    ks = lax.broadcasted_iota(jnp.int32, (1, K), 1)

    def _col(x, zero):
        return jnp.sum(jnp.where(eye, jnp.broadcast_to(x, KK), zero),
                       axis=1, keepdims=True, dtype=x.dtype)

    total = f32(0.0)
    n_valid = f32(0.0)

    for b in range(B):
        c2col = jnp.sum(counts_ref[b], axis=1, keepdims=True)
        c2 = jnp.sum(jnp.where(eye, jnp.broadcast_to(c2col, KK), 0.0),
                     axis=0, keepdims=True)
        s2 = sums_ref[b]
        mx = jnp.max(jnp.where(c2 > 0, jnp.bitwise_and(ks, 31), 0))
        mx_safe = jnp.maximum(mx, 1)
        keyed = mx * jnp.right_shift(row, 5) + jnp.bitwise_and(row, 31)
        remap = (keyed == col).astype(f32)
        counts = jnp.dot(c2, remap, preferred_element_type=f32)
        sums = jnp.dot(s2, remap, preferred_element_type=f32)
        means = sums / jnp.maximum(counts, 1.0)

        present = counts > 0
        n_present = jnp.sum(present.astype(f32))

        P = jnp.zeros(KK, f32)
        for db in range(D // 8):
            md = means[db * 8:(db + 1) * 8, :]
            diff = jnp.abs(md[:, :, None] - md[:, None, :])
            P = P + jnp.sum(diff, axis=0)
        P = P * f32(1.0 / D)

        pk = jnp.where(present, ks, -1)
        k_last = jnp.max(pk)
        k_prev = jnp.max(jnp.where(ks == k_last, -1, pk))
        mxf = mx_safe.astype(f32)
        cls_f = jnp.floor((ks.astype(f32) - 1.0) / mxf)
        eff = jnp.where(ks == k_last,
                        jnp.floor((k_prev.astype(f32) - 1.0) / mxf),
                        cls_f)
        included = present & (counts >= 2.0) & (ks >= 1)
        incval = jnp.where(included, ks, -1)
        inc_col = _col(incval, 0)
        strict = row < col
        prev_idx = jnp.max(jnp.where(strict, jnp.broadcast_to(inc_col, KK), -1),
                           axis=0, keepdims=True)
        pidx = jnp.maximum(prev_idx, 0)
        eff_col = _col(eff, f32(0.0))
        gmat = row == jnp.broadcast_to(pidx, KK)
        prev_cls = jnp.sum(jnp.where(gmat, jnp.broadcast_to(eff_col, KK), 0.0),
                           axis=0, keepdims=True)
        prev_cls = jnp.where(prev_idx < 0, f32(-1e9), prev_cls)
        new_group = included & (eff != prev_cls)
        ng_col = _col(new_group.astype(f32), f32(0.0))
        lower = row <= col
        cums = jnp.sum(jnp.where(lower, jnp.broadcast_to(ng_col, KK), 0.0),
                       axis=0, keepdims=True)
        gid = jnp.where(included, cums - 1.0, f32(-1.0))
        memb = (row.astype(f32) == jnp.broadcast_to(gid, KK)).astype(f32)
        sizes = jnp.sum(memb, axis=1, keepdims=True)
        mp = jnp.dot(memb, P, preferred_element_type=f32)
        num = jnp.sum(mp * memb, axis=1, keepdims=True)
        ret = num / (jnp.maximum(sizes, 1.0) ** 2)
        ret = jnp.where(ret < 1.0, 0.5 * ret * ret, ret - 0.5)
        validg = (sizes > 0.0) & (n_present > 1.0)
        total = total + jnp.sum(jnp.where(validg, ret, 0.0))
        n_valid = n_valid + jnp.sum(validg.astype(f32))

    loss = jnp.where(n_valid > 0.0, total / jnp.maximum(n_valid, 1.0), 0.0)
    out_ref[...] = jnp.broadcast_to(loss * f32(1.0 / B), (1, 1))


def kernel(feature_out, labels, indexes):
    lab = labels.astype(jnp.int32)
    idx = indexes.astype(jnp.int32)
    sums, counts = pl.pallas_call(
        _segsum_body,
        grid=(B, NB),
        in_specs=[
            pl.BlockSpec((1, 8, 384), lambda b, j: (b, j, _i0())),
            pl.BlockSpec((1, 8, 384), lambda b, j: (b, j, _i0())),
            pl.BlockSpec((1, D, 8, 384), lambda b, j: (b, _i0(), j, _i0())),
        ],
        out_specs=[
            pl.BlockSpec((1, D, K), lambda b, j: (b, _i0(), _i0())),
            pl.BlockSpec((1, K, 128), lambda b, j: (b, _i0(), _i0())),
        ],
        out_shape=[
            jax.ShapeDtypeStruct((B, D, K), jnp.float32),
            jax.ShapeDtypeStruct((B, K, 128), jnp.float32),
        ],
        compiler_params=pltpu.CompilerParams(
            dimension_semantics=("arbitrary", "arbitrary")),
    )(lab, idx, feature_out)
    loss = pl.pallas_call(
        _finalize_body,
        out_shape=jax.ShapeDtypeStruct((1, 1), jnp.float32),
    )(sums, counts)
    return loss.reshape(1)


# speedup vs baseline: 1.3028x; 1.2101x over previous
import jax
import jax.numpy as jnp
from jax import lax
from jax.experimental import pallas as pl
from jax.experimental.pallas import tpu as pltpu

N_CLASSES = 5
K = 32 * N_CLASSES
D = 192
HW = 384 * 384
B = 2
PB = 6144
NB = HW // PB
IGNORE_LB = 255


def _i0():
    return jnp.int32(0)


def _segsum_body(lab_ref, idx_ref, feat_ref, sums_ref, counts_ref):
    j = pl.program_id(1)

    @pl.when(j == 0)
    def _init():
        sums_ref[...] = jnp.zeros_like(sums_ref)
        counts_ref[...] = jnp.zeros_like(counts_ref)

    lab = lab_ref[0].reshape(1, PB)
    idx = idx_ref[0].reshape(1, PB)
    k2 = jnp.where(lab == IGNORE_LB, 0, lab * 32 + idx)
    rows = lax.broadcasted_iota(jnp.int32, (K, PB), 0)
    oh_t = (rows == jnp.broadcast_to(k2, (K, PB))).astype(jnp.float32)
    feat = feat_ref[0].reshape(D, PB)
    dn = (((1,), (1,)), ((), ()))
    acc = lax.dot_general(feat, oh_t, dn, preferred_element_type=jnp.float32)
    sums_ref[...] += acc[None]
    cnt = oh_t[:, 0:128]
    for c in range(1, PB // 128):
        cnt = cnt + oh_t[:, c * 128:(c + 1) * 128]
    counts_ref[...] += cnt[None]


def _finalize_body(sums_ref, counts_ref, out_ref):
    f32 = jnp.float32
    KK = (K, K)
    row = lax.broadcasted_iota(jnp.int32, KK, 0)
    col = lax.broadcasted_iota(jnp.int32, KK, 1)
    eye = row == col
    ks = lax.broadcasted_iota(jnp.int32, (1, K), 1)

    def _col(x, zero):
        return jnp.sum(jnp.where(eye, jnp.broadcast_to(x, KK), zero),
                       axis=1, keepdims=True, dtype=x.dtype)

    total = f32(0.0)
    n_valid = f32(0.0)

    for b in range(B):
        c2col = jnp.sum(counts_ref[b], axis=1, keepdims=True)
        c2 = jnp.sum(jnp.where(eye, jnp.broadcast_to(c2col, KK), 0.0),
                     axis=0, keepdims=True)
        s2 = sums_ref[b]
        mx = jnp.max(jnp.where(c2 > 0, jnp.bitwise_and(ks, 31), 0))
        mx_safe = jnp.maximum(mx, 1)
        keyed = mx * jnp.right_shift(row, 5) + jnp.bitwise_and(row, 31)
        remap = (keyed == col).astype(f32)
        counts = jnp.dot(c2, remap, preferred_element_type=f32)
        sums = jnp.dot(s2, remap, preferred_element_type=f32)
        means = sums / jnp.maximum(counts, 1.0)

        present = counts > 0
        n_present = jnp.sum(present.astype(f32))

        P = jnp.zeros(KK, f32)
        for db in range(D // 8):
            md = means[db * 8:(db + 1) * 8, :]
            diff = jnp.abs(md[:, :, None] - md[:, None, :])
            P = P + jnp.sum(diff, axis=0)
        P = P * f32(1.0 / D)

        pk = jnp.where(present, ks, -1)
        k_last = jnp.max(pk)
        k_prev = jnp.max(jnp.where(ks == k_last, -1, pk))
        mxf = mx_safe.astype(f32)
        cls_f = jnp.floor((ks.astype(f32) - 1.0) / mxf)
        eff = jnp.where(ks == k_last,
                        jnp.floor((k_prev.astype(f32) - 1.0) / mxf),
                        cls_f)
        included = present & (counts >= 2.0) & (ks >= 1)
        incval = jnp.where(included, ks, -1)
        inc_col = _col(incval, 0)
        strict = row < col
        prev_idx = jnp.max(jnp.where(strict, jnp.broadcast_to(inc_col, KK), -1),
                           axis=0, keepdims=True)
        pidx = jnp.maximum(prev_idx, 0)
        eff_col = _col(eff, f32(0.0))
        gmat = row == jnp.broadcast_to(pidx, KK)
        prev_cls = jnp.sum(jnp.where(gmat, jnp.broadcast_to(eff_col, KK), 0.0),
                           axis=0, keepdims=True)
        prev_cls = jnp.where(prev_idx < 0, f32(-1e9), prev_cls)
        new_group = included & (eff != prev_cls)
        ng_col = _col(new_group.astype(f32), f32(0.0))
        lower = row <= col
        cums = jnp.sum(jnp.where(lower, jnp.broadcast_to(ng_col, KK), 0.0),
                       axis=0, keepdims=True)
        gid = jnp.where(included, cums - 1.0, f32(-1.0))
        memb = (row.astype(f32) == jnp.broadcast_to(gid, KK)).astype(f32)
        sizes = jnp.sum(memb, axis=1, keepdims=True)
        mp = jnp.dot(memb, P, preferred_element_type=f32)
        num = jnp.sum(mp * memb, axis=1, keepdims=True)
        ret = num / (jnp.maximum(sizes, 1.0) ** 2)
        ret = jnp.where(ret < 1.0, 0.5 * ret * ret, ret - 0.5)
        validg = (sizes > 0.0) & (n_present > 1.0)
        total = total + jnp.sum(jnp.where(validg, ret, 0.0))
        n_valid = n_valid + jnp.sum(validg.astype(f32))

    loss = jnp.where(n_valid > 0.0, total / jnp.maximum(n_valid, 1.0), 0.0)
    out_ref[...] = jnp.broadcast_to(loss * f32(1.0 / B), (1, 1))


def kernel(feature_out, labels, indexes):
    lab = labels.astype(jnp.int32)
    idx = indexes.astype(jnp.int32)
    sums, counts = pl.pallas_call(
        _segsum_body,
        grid=(B, NB),
        in_specs=[
            pl.BlockSpec((1, 16, 384), lambda b, j: (b, j, _i0())),
            pl.BlockSpec((1, 16, 384), lambda b, j: (b, j, _i0())),
            pl.BlockSpec((1, D, 16, 384), lambda b, j: (b, _i0(), j, _i0())),
        ],
        out_specs=[
            pl.BlockSpec((1, D, K), lambda b, j: (b, _i0(), _i0())),
            pl.BlockSpec((1, K, 128), lambda b, j: (b, _i0(), _i0())),
        ],
        out_shape=[
            jax.ShapeDtypeStruct((B, D, K), jnp.float32),
            jax.ShapeDtypeStruct((B, K, 128), jnp.float32),
        ],
        compiler_params=pltpu.CompilerParams(
            dimension_semantics=("arbitrary", "arbitrary")),
    )(lab, idx, feature_out)
    loss = pl.pallas_call(
        _finalize_body,
        out_shape=jax.ShapeDtypeStruct((1, 1), jnp.float32),
    )(sums, counts)
    return loss.reshape(1)


# speedup vs baseline: 1.4475x; 1.1111x over previous
import jax
import jax.numpy as jnp
from jax import lax
from jax.experimental import pallas as pl
from jax.experimental.pallas import tpu as pltpu

N_CLASSES = 5
K = 32 * N_CLASSES
D = 192
HW = 384 * 384
B = 2
PB = 12288
NB = HW // PB
IGNORE_LB = 255


def _i0():
    return jnp.int32(0)


def _segsum_body(lab_ref, idx_ref, feat_ref, sums_ref, counts_ref):
    j = pl.program_id(1)

    @pl.when(j == 0)
    def _init():
        sums_ref[...] = jnp.zeros_like(sums_ref)
        counts_ref[...] = jnp.zeros_like(counts_ref)

    lab = lab_ref[0].reshape(1, PB)
    idx = idx_ref[0].reshape(1, PB)
    k2 = jnp.where(lab == IGNORE_LB, 0, lab * 32 + idx)
    rows = lax.broadcasted_iota(jnp.int32, (K, PB), 0)
    oh_t = (rows == jnp.broadcast_to(k2, (K, PB))).astype(jnp.float32)
    feat = feat_ref[0].reshape(D, PB)
    dn = (((1,), (1,)), ((), ()))
    acc = lax.dot_general(feat, oh_t, dn, preferred_element_type=jnp.float32)
    sums_ref[...] += acc[None]
    cnt = oh_t[:, 0:128]
    for c in range(1, PB // 128):
        cnt = cnt + oh_t[:, c * 128:(c + 1) * 128]
    counts_ref[...] += cnt[None]


def _finalize_body(sums_ref, counts_ref, out_ref):
    f32 = jnp.float32
    KK = (K, K)
    row = lax.broadcasted_iota(jnp.int32, KK, 0)
    col = lax.broadcasted_iota(jnp.int32, KK, 1)
    eye = row == col
    ks = lax.broadcasted_iota(jnp.int32, (1, K), 1)

    def _col(x, zero):
        return jnp.sum(jnp.where(eye, jnp.broadcast_to(x, KK), zero),
                       axis=1, keepdims=True, dtype=x.dtype)

    total = f32(0.0)
    n_valid = f32(0.0)

    for b in range(B):
        c2col = jnp.sum(counts_ref[b], axis=1, keepdims=True)
        c2 = jnp.sum(jnp.where(eye, jnp.broadcast_to(c2col, KK), 0.0),
                     axis=0, keepdims=True)
        s2 = sums_ref[b]
        mx = jnp.max(jnp.where(c2 > 0, jnp.bitwise_and(ks, 31), 0))
        mx_safe = jnp.maximum(mx, 1)
        keyed = mx * jnp.right_shift(row, 5) + jnp.bitwise_and(row, 31)
        remap = (keyed == col).astype(f32)
        counts = jnp.dot(c2, remap, preferred_element_type=f32)
        sums = jnp.dot(s2, remap, preferred_element_type=f32)
        means = sums / jnp.maximum(counts, 1.0)

        present = counts > 0
        n_present = jnp.sum(present.astype(f32))

        P = jnp.zeros(KK, f32)
        for db in range(D // 8):
            md = means[db * 8:(db + 1) * 8, :]
            diff = jnp.abs(md[:, :, None] - md[:, None, :])
            P = P + jnp.sum(diff, axis=0)
        P = P * f32(1.0 / D)

        pk = jnp.where(present, ks, -1)
        k_last = jnp.max(pk)
        k_prev = jnp.max(jnp.where(ks == k_last, -1, pk))
        mxf = mx_safe.astype(f32)
        cls_f = jnp.floor((ks.astype(f32) - 1.0) / mxf)
        eff = jnp.where(ks == k_last,
                        jnp.floor((k_prev.astype(f32) - 1.0) / mxf),
                        cls_f)
        included = present & (counts >= 2.0) & (ks >= 1)
        incval = jnp.where(included, ks, -1)
        inc_col = _col(incval, 0)
        strict = row < col
        prev_idx = jnp.max(jnp.where(strict, jnp.broadcast_to(inc_col, KK), -1),
                           axis=0, keepdims=True)
        pidx = jnp.maximum(prev_idx, 0)
        eff_col = _col(eff, f32(0.0))
        gmat = row == jnp.broadcast_to(pidx, KK)
        prev_cls = jnp.sum(jnp.where(gmat, jnp.broadcast_to(eff_col, KK), 0.0),
                           axis=0, keepdims=True)
        prev_cls = jnp.where(prev_idx < 0, f32(-1e9), prev_cls)
        new_group = included & (eff != prev_cls)
        ng_col = _col(new_group.astype(f32), f32(0.0))
        lower = row <= col
        cums = jnp.sum(jnp.where(lower, jnp.broadcast_to(ng_col, KK), 0.0),
                       axis=0, keepdims=True)
        gid = jnp.where(included, cums - 1.0, f32(-1.0))
        memb = (row.astype(f32) == jnp.broadcast_to(gid, KK)).astype(f32)
        sizes = jnp.sum(memb, axis=1, keepdims=True)
        mp = jnp.dot(memb, P, preferred_element_type=f32)
        num = jnp.sum(mp * memb, axis=1, keepdims=True)
        ret = num / (jnp.maximum(sizes, 1.0) ** 2)
        ret = jnp.where(ret < 1.0, 0.5 * ret * ret, ret - 0.5)
        validg = (sizes > 0.0) & (n_present > 1.0)
        total = total + jnp.sum(jnp.where(validg, ret, 0.0))
        n_valid = n_valid + jnp.sum(validg.astype(f32))

    loss = jnp.where(n_valid > 0.0, total / jnp.maximum(n_valid, 1.0), 0.0)
    out_ref[...] = jnp.broadcast_to(loss * f32(1.0 / B), (1, 1))


def kernel(feature_out, labels, indexes):
    lab = labels.astype(jnp.int32)
    idx = indexes.astype(jnp.int32)
    sums, counts = pl.pallas_call(
        _segsum_body,
        grid=(B, NB),
        in_specs=[
            pl.BlockSpec((1, 32, 384), lambda b, j: (b, j, _i0())),
            pl.BlockSpec((1, 32, 384), lambda b, j: (b, j, _i0())),
            pl.BlockSpec((1, D, 32, 384), lambda b, j: (b, _i0(), j, _i0())),
        ],
        out_specs=[
            pl.BlockSpec((1, D, K), lambda b, j: (b, _i0(), _i0())),
            pl.BlockSpec((1, K, 128), lambda b, j: (b, _i0(), _i0())),
        ],
        out_shape=[
            jax.ShapeDtypeStruct((B, D, K), jnp.float32),
            jax.ShapeDtypeStruct((B, K, 128), jnp.float32),
        ],
        compiler_params=pltpu.CompilerParams(
            dimension_semantics=("arbitrary", "arbitrary")),
    )(lab, idx, feature_out)
    loss = pl.pallas_call(
        _finalize_body,
        out_shape=jax.ShapeDtypeStruct((1, 1), jnp.float32),
    )(sums, counts)
    return loss.reshape(1)


# speedup vs baseline: 1.5079x; 1.0418x over previous
import jax
import jax.numpy as jnp
from jax import lax
from jax.experimental import pallas as pl
from jax.experimental.pallas import tpu as pltpu

N_CLASSES = 5
K = 32 * N_CLASSES
D = 192
HW = 384 * 384
B = 2
PB = 18432
NB = HW // PB
IGNORE_LB = 255


def _i0():
    return jnp.int32(0)


def _segsum_body(lab_ref, idx_ref, feat_ref, sums_ref, counts_ref):
    j = pl.program_id(1)

    @pl.when(j == 0)
    def _init():
        sums_ref[...] = jnp.zeros_like(sums_ref)
        counts_ref[...] = jnp.zeros_like(counts_ref)

    lab = lab_ref[0].reshape(1, PB)
    idx = idx_ref[0].reshape(1, PB)
    k2 = jnp.where(lab == IGNORE_LB, 0, lab * 32 + idx)
    rows = lax.broadcasted_iota(jnp.int32, (K, PB), 0)
    oh_t = (rows == jnp.broadcast_to(k2, (K, PB))).astype(jnp.float32)
    feat = feat_ref[0].reshape(D, PB)
    dn = (((1,), (1,)), ((), ()))
    acc = lax.dot_general(feat, oh_t, dn, preferred_element_type=jnp.float32)
    sums_ref[...] += acc[None]
    cnt = oh_t[:, 0:128]
    for c in range(1, PB // 128):
        cnt = cnt + oh_t[:, c * 128:(c + 1) * 128]
    counts_ref[...] += cnt[None]


def _finalize_body(sums_ref, counts_ref, out_ref):
    f32 = jnp.float32
    KK = (K, K)
    row = lax.broadcasted_iota(jnp.int32, KK, 0)
    col = lax.broadcasted_iota(jnp.int32, KK, 1)
    eye = row == col
    ks = lax.broadcasted_iota(jnp.int32, (1, K), 1)

    def _col(x, zero):
        return jnp.sum(jnp.where(eye, jnp.broadcast_to(x, KK), zero),
                       axis=1, keepdims=True, dtype=x.dtype)

    total = f32(0.0)
    n_valid = f32(0.0)

    for b in range(B):
        c2col = jnp.sum(counts_ref[b], axis=1, keepdims=True)
        c2 = jnp.sum(jnp.where(eye, jnp.broadcast_to(c2col, KK), 0.0),
                     axis=0, keepdims=True)
        s2 = sums_ref[b]
        mx = jnp.max(jnp.where(c2 > 0, jnp.bitwise_and(ks, 31), 0))
        mx_safe = jnp.maximum(mx, 1)
        keyed = mx * jnp.right_shift(row, 5) + jnp.bitwise_and(row, 31)
        remap = (keyed == col).astype(f32)
        counts = jnp.dot(c2, remap, preferred_element_type=f32)
        sums = jnp.dot(s2, remap, preferred_element_type=f32)
        means = sums / jnp.maximum(counts, 1.0)

        present = counts > 0
        n_present = jnp.sum(present.astype(f32))

        P = jnp.zeros(KK, f32)
        for db in range(D // 8):
            md = means[db * 8:(db + 1) * 8, :]
            diff = jnp.abs(md[:, :, None] - md[:, None, :])
            P = P + jnp.sum(diff, axis=0)
        P = P * f32(1.0 / D)

        pk = jnp.where(present, ks, -1)
        k_last = jnp.max(pk)
        k_prev = jnp.max(jnp.where(ks == k_last, -1, pk))
        mxf = mx_safe.astype(f32)
        cls_f = jnp.floor((ks.astype(f32) - 1.0) / mxf)
        eff = jnp.where(ks == k_last,
                        jnp.floor((k_prev.astype(f32) - 1.0) / mxf),
                        cls_f)
        included = present & (counts >= 2.0) & (ks >= 1)
        incval = jnp.where(included, ks, -1)
        inc_col = _col(incval, 0)
        strict = row < col
        prev_idx = jnp.max(jnp.where(strict, jnp.broadcast_to(inc_col, KK), -1),
                           axis=0, keepdims=True)
        pidx = jnp.maximum(prev_idx, 0)
        eff_col = _col(eff, f32(0.0))
        gmat = row == jnp.broadcast_to(pidx, KK)
        prev_cls = jnp.sum(jnp.where(gmat, jnp.broadcast_to(eff_col, KK), 0.0),
                           axis=0, keepdims=True)
        prev_cls = jnp.where(prev_idx < 0, f32(-1e9), prev_cls)
        new_group = included & (eff != prev_cls)
        ng_col = _col(new_group.astype(f32), f32(0.0))
        lower = row <= col
        cums = jnp.sum(jnp.where(lower, jnp.broadcast_to(ng_col, KK), 0.0),
                       axis=0, keepdims=True)
        gid = jnp.where(included, cums - 1.0, f32(-1.0))
        memb = (row.astype(f32) == jnp.broadcast_to(gid, KK)).astype(f32)
        sizes = jnp.sum(memb, axis=1, keepdims=True)
        mp = jnp.dot(memb, P, preferred_element_type=f32)
        num = jnp.sum(mp * memb, axis=1, keepdims=True)
        ret = num / (jnp.maximum(sizes, 1.0) ** 2)
        ret = jnp.where(ret < 1.0, 0.5 * ret * ret, ret - 0.5)
        validg = (sizes > 0.0) & (n_present > 1.0)
        total = total + jnp.sum(jnp.where(validg, ret, 0.0))
        n_valid = n_valid + jnp.sum(validg.astype(f32))

    loss = jnp.where(n_valid > 0.0, total / jnp.maximum(n_valid, 1.0), 0.0)
    out_ref[...] = jnp.broadcast_to(loss * f32(1.0 / B), (1, 1))


def kernel(feature_out, labels, indexes):
    lab = labels.astype(jnp.int32)
    idx = indexes.astype(jnp.int32)
    sums, counts = pl.pallas_call(
        _segsum_body,
        grid=(B, NB),
        in_specs=[
            pl.BlockSpec((1, 48, 384), lambda b, j: (b, j, _i0())),
            pl.BlockSpec((1, 48, 384), lambda b, j: (b, j, _i0())),
            pl.BlockSpec((1, D, 48, 384), lambda b, j: (b, _i0(), j, _i0())),
        ],
        out_specs=[
            pl.BlockSpec((1, D, K), lambda b, j: (b, _i0(), _i0())),
            pl.BlockSpec((1, K, 128), lambda b, j: (b, _i0(), _i0())),
        ],
        out_shape=[
            jax.ShapeDtypeStruct((B, D, K), jnp.float32),
            jax.ShapeDtypeStruct((B, K, 128), jnp.float32),
        ],
        compiler_params=pltpu.CompilerParams(
            dimension_semantics=("arbitrary", "arbitrary")),
    )(lab, idx, feature_out)
    loss = pl.pallas_call(
        _finalize_body,
        out_shape=jax.ShapeDtypeStruct((1, 1), jnp.float32),
    )(sums, counts)
    return loss.reshape(1)
